# baseline (speedup 1.0000x reference)
import functools

import jax
import jax.numpy as jnp
from jax import lax
from jax.experimental import pallas as pl
from jax.experimental.pallas import tpu as pltpu
from jax.experimental.pallas import tpu_sc as plsc

D = 64
DP = 128
NB, S = 4096, 200
B = NB * S
NC, NS = 2, 16
NW = NC * NS
BPW = B // NW
RPW = NB // NW
RPC = 2
CHUNK = RPC * S
NCHUNK = RPW // RPC
GSPLIT = ((0, 128), (128, 72))


def _build():
    mesh = plsc.VectorSubcoreMesh(core_axis_name="c", subcore_axis_name="s")

    @functools.partial(
        pl.kernel,
        mesh=mesh,
        out_type=jax.ShapeDtypeStruct((NB, S, DP), jnp.float32),
        scratch_types=[
            pltpu.VMEM((BPW,), jnp.int32),
            pltpu.VMEM((RPC, S, DP), jnp.float32),
            pltpu.VMEM((RPC, S, DP), jnp.float32),
            pltpu.SemaphoreType.DMA,
            pltpu.SemaphoreType.DMA,
        ],
    )
    def gather_kernel(nodes_hbm, table_hbm, out_hbm, idx_v, rows0, rows1,
                      gsem, osem):
        cid = lax.axis_index("c")
        sid = lax.axis_index("s")
        wid = sid * NC + cid
        base = wid * BPW
        rbase = wid * RPW

        pltpu.sync_copy(nodes_hbm.at[pl.ds(base, BPW)], idx_v)

        def drain(rows, sem):
            pltpu.make_async_copy(out_hbm.at[pl.ds(0, RPC)], rows, sem).wait()

        def half_step(g, rows):
            @pl.when(g >= 2)
            def _():
                drain(rows, osem)
            for r in range(RPC):
                for (o, w) in GSPLIT:
                    pltpu.async_copy(
                        table_hbm.at[idx_v.at[pl.ds(g * CHUNK + r * S + o, w)]],
                        rows.at[r].at[pl.ds(o, w)],
                        gsem,
                    )
            drain(rows, gsem)
            pltpu.async_copy(rows, out_hbm.at[pl.ds(rbase + g * RPC, RPC)],
                             osem)

        def body(j, carry):
            half_step(2 * j, rows0)
            half_step(2 * j + 1, rows1)
            return carry

        lax.fori_loop(0, NCHUNK // 2, body, 0)
        drain(rows0, osem)
        drain(rows1, osem)

    return gather_kernel


_GATHER = _build()


def kernel(nodes, table):
    nodes_flat = jnp.asarray(nodes, jnp.int32).reshape(B)
    table_p = jnp.pad(table, ((0, 0), (0, DP - D)))
    out = _GATHER(nodes_flat, table_p)
    return jnp.where((nodes == 0)[..., None], jnp.float32(0), out[:, :, :D])
